# Initial kernel scaffold; baseline (speedup 1.0000x reference)
import jax
import jax.numpy as jnp
from jax.experimental import pallas as pl


def kernel(x, segment_ids):
    raise NotImplementedError("write your pallas kernel here")


# speedup vs baseline: 2.5859x; 2.5859x over previous
import functools

import jax
import jax.numpy as jnp
from jax import lax
from jax.experimental import pallas as pl
from jax.experimental.pallas import tpu as pltpu
from jax.experimental.pallas import tpu_sc as plsc

N_EDGES = 320000
D = 128
S = 10000
SP = 10240

NC = 2
NS = 16
NW = NC * NS
EPW = N_EDGES // NW
CH = 80
NCHUNK = EPW // CH
ZROWS = 128
SPT = SP // NS


def _sc_body(x_hbm, ids_hbm, part_hbm, idx_v, rows_v, zeros_v, acc_sh):
    c = lax.axis_index("c")
    s = lax.axis_index("s")
    wid = c * NS + s

    def zero_store(k, _):
        i = k // (D // 16)
        j = k % (D // 16)
        zeros_v[i, pl.ds(j * 16, 16)] = jnp.zeros((16,), jnp.float32)
        return 0
    lax.fori_loop(0, ZROWS * (D // 16), zero_store, 0)

    for r in range(SPT // ZROWS):
        pltpu.sync_copy(zeros_v, acc_sh.at[pl.ds(s * SPT + r * ZROWS, ZROWS)])
    plsc.subcore_barrier()

    base = wid * EPW

    def chunk(i, _):
        off = base + i * CH
        pltpu.sync_copy(ids_hbm.at[pl.ds(off, CH)], idx_v)
        pltpu.sync_copy(x_hbm.at[pl.ds(off, CH)], rows_v)
        pltpu.sync_copy(rows_v, acc_sh.at[idx_v], add=True)
        return 0
    lax.fori_loop(0, NCHUNK, chunk, 0)
    plsc.subcore_barrier()

    pltpu.sync_copy(acc_sh.at[pl.ds(s * SPT, SPT)],
                    part_hbm.at[c, pl.ds(s * SPT, SPT)])


@functools.cache
def _sc_call():
    return pl.kernel(
        _sc_body,
        out_type=jax.ShapeDtypeStruct((NC, SP, D), jnp.float32),
        mesh=plsc.VectorSubcoreMesh(core_axis_name="c", subcore_axis_name="s",
                                    num_cores=NC, num_subcores=NS),
        scratch_types=[
            pltpu.VMEM((CH,), jnp.int32),
            pltpu.VMEM((CH, D), jnp.float32),
            pltpu.VMEM((ZROWS, D), jnp.float32),
            pltpu.VMEM_SHARED((SP, D), jnp.float32),
        ],
    )


def _add_body(p_ref, o_ref):
    o_ref[...] = p_ref[0] + p_ref[1]


_ROWS_PER_BLK = 1000


def _combine(partials):
    return pl.pallas_call(
        _add_body,
        grid=(S // _ROWS_PER_BLK,),
        in_specs=[pl.BlockSpec((NC, _ROWS_PER_BLK, D), lambda i: (0, i, 0))],
        out_specs=pl.BlockSpec((_ROWS_PER_BLK, D), lambda i: (i, 0)),
        out_shape=jax.ShapeDtypeStruct((S, D), jnp.float32),
    )(partials)


def kernel(x, segment_ids):
    ids32 = segment_ids.astype(jnp.int32)
    partials = _sc_call()(x, ids32)
    return _combine(partials)


# speedup vs baseline: 5.8091x; 2.2464x over previous
import functools

import jax
import jax.numpy as jnp
from jax import lax
from jax.experimental import pallas as pl
from jax.experimental.pallas import tpu as pltpu
from jax.experimental.pallas import tpu_sc as plsc

N_EDGES = 320000
D = 128
S = 10000
SP = 10240

NC = 2
NS = 16
NW = NC * NS
EPW = N_EDGES // NW
CH = 80
NCHUNK = EPW // CH
NBUF = 4
NGRP = NCHUNK // NBUF
ZROWS = 32
SPT = SP // NS


def _sc_body(x_hbm, ids_hbm, part_hbm, idx_v, rows_v, zeros_v, acc_sh,
             s0, s1, s2, s3, zsem):
    sems = (s0, s1, s2, s3)
    c = lax.axis_index("c")
    s = lax.axis_index("s")
    wid = c * NS + s
    base = wid * EPW

    def start_load(i, b):
        off = base + i * CH
        pltpu.async_copy(ids_hbm.at[pl.ds(off, CH)], idx_v.at[b], sems[b])
        pltpu.async_copy(x_hbm.at[pl.ds(off, CH)], rows_v.at[b], sems[b])

    def wait_load(i, b):
        off = base + i * CH
        pltpu.make_async_copy(ids_hbm.at[pl.ds(off, CH)], idx_v.at[b],
                              sems[b]).wait()
        pltpu.make_async_copy(x_hbm.at[pl.ds(off, CH)], rows_v.at[b],
                              sems[b]).wait()

    for b in range(NBUF):
        start_load(b, b)

    def zero_store(k, _):
        i = k // (D // 16)
        j = k % (D // 16)
        zeros_v[i, pl.ds(j * 16, 16)] = jnp.zeros((16,), jnp.float32)
        return 0
    lax.fori_loop(0, ZROWS * (D // 16), zero_store, 0)

    def zero_start(r, _):
        pltpu.async_copy(zeros_v, acc_sh.at[pl.ds(s * SPT + r * ZROWS, ZROWS)],
                         zsem)
        return 0
    lax.fori_loop(0, SPT // ZROWS, zero_start, 0)

    def zero_wait(r, _):
        pltpu.make_async_copy(
            zeros_v, acc_sh.at[pl.ds(s * SPT + r * ZROWS, ZROWS)], zsem).wait()
        return 0
    lax.fori_loop(0, SPT // ZROWS, zero_wait, 0)
    plsc.subcore_barrier()

    def group(g, _):
        for b in range(NBUF):
            i = g * NBUF + b
            wait_load(i, b)
            pltpu.sync_copy(rows_v.at[b], acc_sh.at[idx_v.at[b]], add=True)
            ip = i + NBUF

            @pl.when(ip < NCHUNK)
            def _():
                start_load(ip, b)
        return 0
    lax.fori_loop(0, NGRP, group, 0)

    for i in range(NGRP * NBUF, NCHUNK):
        b = i % NBUF
        wait_load(i, b)
        pltpu.sync_copy(rows_v.at[b], acc_sh.at[idx_v.at[b]], add=True)
    plsc.subcore_barrier()

    pltpu.sync_copy(acc_sh.at[pl.ds(s * SPT, SPT)],
                    part_hbm.at[c, pl.ds(s * SPT, SPT)])


@functools.cache
def _sc_call():
    return pl.kernel(
        _sc_body,
        out_type=jax.ShapeDtypeStruct((NC, SP, D), jnp.float32),
        mesh=plsc.VectorSubcoreMesh(core_axis_name="c", subcore_axis_name="s",
                                    num_cores=NC, num_subcores=NS),
        scratch_types=[
            pltpu.VMEM((NBUF, CH), jnp.int32),
            pltpu.VMEM((NBUF, CH, D), jnp.float32),
            pltpu.VMEM((ZROWS, D), jnp.float32),
            pltpu.VMEM_SHARED((SP, D), jnp.float32),
        ] + [pltpu.SemaphoreType.DMA] * (NBUF + 1),
    )


def _add_body(p_ref, o_ref):
    o_ref[...] = p_ref[0] + p_ref[1]


_ROWS_PER_BLK = 1000


def _combine(partials):
    return pl.pallas_call(
        _add_body,
        grid=(S // _ROWS_PER_BLK,),
        in_specs=[pl.BlockSpec((NC, _ROWS_PER_BLK, D), lambda i: (0, i, 0))],
        out_specs=pl.BlockSpec((_ROWS_PER_BLK, D), lambda i: (i, 0)),
        out_shape=jax.ShapeDtypeStruct((S, D), jnp.float32),
    )(partials)


def kernel(x, segment_ids):
    ids32 = segment_ids.astype(jnp.int32)
    partials = _sc_call()(x, ids32)
    return _combine(partials)
